# baseline (speedup 1.0000x reference)
import functools
import math

import jax
import jax.numpy as jnp
from jax import lax
from jax.experimental import pallas as pl
from jax.experimental.pallas import tpu as pltpu
from jax.experimental.pallas import tpu_sc as plsc

HIDDEN = 2048
INTER = 8192
K = 8
TOKENS = 4 * 2048
N_CHUNKS = 2


T_BLK = 2048
I_BLK = 256
N_TILES = INTER // I_BLK
N_CAND = N_TILES * K
_INV_SQRT2 = 0.7071067811865476
_NEG_INF = float("-inf")


def _tc_body(x_ref, w_ref, b_ref, cv_ref, ci_ref):
    j = pl.program_id(1)

    h = jnp.dot(x_ref[...], w_ref[...], preferred_element_type=jnp.float32)
    h = h + b_ref[...]
    h = 0.5 * h * (1.0 + lax.erf(h * _INV_SQRT2))

    iota = lax.broadcasted_iota(jnp.int32, (T_BLK, I_BLK), 1).astype(
        jnp.float32)
    base = (j * I_BLK).astype(jnp.float32)
    cur = h
    vs, ids = [], []
    for _ in range(K):
        m = jnp.max(cur, axis=1, keepdims=True)
        idx = jnp.min(jnp.where(cur == m, iota, float(I_BLK)), axis=1,
                      keepdims=True)
        vs.append(m)
        ids.append(idx + base)
        cur = jnp.where(iota == idx, _NEG_INF, cur)
    cv_ref[...] = jnp.concatenate(vs, axis=1)[None]
    ci_ref[...] = jnp.concatenate(ids, axis=1)[None]


def _sel_body(cv_ref, ci_ref, g_ref, i_ref):
    cv = cv_ref[...]
    ci = ci_ref[...]
    gs, iis = [], []
    for _ in range(K):
        m = jnp.max(cv, axis=1, keepdims=True)
        idx = jnp.min(jnp.where(cv == m, ci, float(INTER)), axis=1,
                      keepdims=True)
        gs.append(m)
        iis.append(idx)
        cv = jnp.where(ci == idx, _NEG_INF, cv)
    g_ref[...] = jnp.concatenate(gs, axis=1)
    i_ref[...] = jnp.concatenate(iis, axis=1).astype(jnp.int32)


def _topk_call(x2d, W1, b2d):
    n_tok = x2d.shape[0]
    cv, ci = pl.pallas_call(
        _tc_body,
        grid=(n_tok // T_BLK, N_TILES),
        in_specs=[
            pl.BlockSpec((T_BLK, HIDDEN), lambda i, j: (i, 0)),
            pl.BlockSpec((HIDDEN, I_BLK), lambda i, j: (0, j)),
            pl.BlockSpec((1, I_BLK), lambda i, j: (0, j)),
        ],
        out_specs=[
            pl.BlockSpec((1, T_BLK, K), lambda i, j: (j, i, 0)),
            pl.BlockSpec((1, T_BLK, K), lambda i, j: (j, i, 0)),
        ],
        out_shape=[
            jax.ShapeDtypeStruct((N_TILES, n_tok, K), jnp.float32),
            jax.ShapeDtypeStruct((N_TILES, n_tok, K), jnp.float32),
        ],
    )(x2d, W1, b2d)
    cv = cv.transpose(1, 0, 2).reshape(n_tok, N_CAND)
    ci = ci.transpose(1, 0, 2).reshape(n_tok, N_CAND)
    return pl.pallas_call(
        _sel_body,
        grid=(n_tok // T_BLK,),
        in_specs=[
            pl.BlockSpec((T_BLK, N_CAND), lambda i: (i, 0)),
            pl.BlockSpec((T_BLK, N_CAND), lambda i: (i, 0)),
        ],
        out_specs=[
            pl.BlockSpec((T_BLK, K), lambda i: (i, 0)),
            pl.BlockSpec((T_BLK, K), lambda i: (i, 0)),
        ],
        out_shape=[
            jax.ShapeDtypeStruct((n_tok, K), jnp.float32),
            jax.ShapeDtypeStruct((n_tok, K), jnp.int32),
        ],
    )(cv, ci)


NC, NS = 2, 16
NW = NC * NS
C_TOK = 2
OUT_TOK = 8
CH_PER_GRP = OUT_TOK // C_TOK


def _make_sc_body(tpw):
    n_grp = tpw // OUT_TOK
    n_ch = tpw // C_TOK

    def _sc_body(idx_hbm, gate_hbm, emb_hbm, out_hbm, idx_v, gate_v, rows_v,
                 out_v, gsem, osem):
        wid = lax.axis_index("s") * NC + lax.axis_index("c")
        base = wid * tpw

        pltpu.sync_copy(idx_hbm.at[pl.ds(base * K, tpw * K)], idx_v)
        pltpu.sync_copy(gate_hbm.at[pl.ds(base * K, tpw * K)], gate_v)

        def start_gather(c, buf):
            pltpu.async_copy(
                emb_hbm.at[idx_v.at[pl.ds(c * C_TOK * K, C_TOK * K)]],
                rows_v.at[buf], gsem.at[buf],
            )

        start_gather(0, 0)

        def group(o, _):
            ob = lax.rem(o, 2)
            @pl.when(o >= 2)
            def _():
                pltpu.make_async_copy(
                    out_v.at[ob], out_hbm.at[pl.ds(0, OUT_TOK)], osem.at[ob]
                ).wait()

            for cc in range(CH_PER_GRP):
                c = o * CH_PER_GRP + cc
                buf = cc % 2

                @pl.when(c + 1 < n_ch)
                def _():
                    start_gather(c + 1, (cc + 1) % 2)

                pltpu.make_async_copy(
                    emb_hbm.at[pl.ds(0, C_TOK * K)], rows_v.at[buf],
                    gsem.at[buf],
                ).wait()

                gvec = gate_v[pl.ds(c * C_TOK * K, C_TOK * K)]
                for t in range(C_TOK):
                    gsc = [gvec[t * K + k] for k in range(K)]
                    row = cc * C_TOK + t

                    @plsc.parallel_loop(0, HIDDEN // 16, unroll=4)
                    def _(d):
                        sl = pl.ds(d * 16, 16)
                        acc = gsc[0] * rows_v[buf, t * K + 0, sl]
                        for k in range(1, K):
                            acc = acc + gsc[k] * rows_v[buf, t * K + k, sl]
                        out_v[ob, row, sl] = acc

            pltpu.async_copy(
                out_v.at[ob], out_hbm.at[pl.ds(base + o * OUT_TOK, OUT_TOK)],
                osem.at[ob],
            )
            return 0

        lax.fori_loop(0, n_grp, group, 0)
        for ob in range(2):
            pltpu.make_async_copy(
                out_v.at[ob], out_hbm.at[pl.ds(0, OUT_TOK)], osem.at[ob]
            ).wait()

    return _sc_body


def _gather_call(idx_flat, gate_flat, emb):
    n_tok = idx_flat.shape[0] // K
    tpw = n_tok // NW
    mesh = plsc.VectorSubcoreMesh(
        core_axis_name="c", subcore_axis_name="s", num_cores=NC,
        num_subcores=NS,
    )
    return pl.kernel(
        _make_sc_body(tpw),
        out_type=jax.ShapeDtypeStruct((n_tok, HIDDEN), jnp.float32),
        mesh=mesh,
        scratch_types=[
            pltpu.VMEM((tpw * K,), jnp.int32),
            pltpu.VMEM((tpw * K,), jnp.float32),
            pltpu.VMEM((2, C_TOK * K, HIDDEN), jnp.float32),
            pltpu.VMEM((2, OUT_TOK, HIDDEN), jnp.float32),
            pltpu.SemaphoreType.DMA((2,)),
            pltpu.SemaphoreType.DMA((2,)),
        ],
    )(idx_flat, gate_flat, emb)


def kernel(x, W1, b1, emb):
    B, S, H = x.shape
    x2d = x.reshape(B * S, H)
    b2d = b1.reshape(1, INTER)
    csz = TOKENS // N_CHUNKS
    outs = []
    for p in range(N_CHUNKS):
        xc = lax.slice_in_dim(x2d, p * csz, (p + 1) * csz, axis=0)
        G, I = _topk_call(xc, W1, b2d)
        outs.append(_gather_call(I.reshape(-1), G.reshape(-1), emb))
    out = jnp.concatenate(outs, axis=0)
    return out.reshape(B, S, H)
